# baseline (speedup 1.0000x reference)
import functools

import jax
import jax.numpy as jnp
from jax import lax
from jax.experimental import pallas as pl
from jax.experimental.pallas import tpu as pltpu
from jax.experimental.pallas import tpu_sc as plsc

B, C, H, W = 2, 96, 384, 384
E = 64
K = 3
T = K * K
N = H * W


_NC, _NS = 2, 16
_NW = _NC * _NS
_PPW = (B * N) // _NW
_WPB = _NW // B
_RPW = _PPW // W

def _sc_wm_body(bank_hbm, bk_hbm, wm_hbm, bank_v, idx_v, wm_v):
    wid = lax.axis_index("s") * _NC + lax.axis_index("c")
    b = wid // _WPB
    off = (wid % _WPB) * _PPW

    pltpu.sync_copy(bank_hbm.at[pl.ds(b * T * E, T * E)], bank_v)
    pltpu.sync_copy(bk_hbm.at[pl.ds(b * N + off, _PPW)], idx_v)

    @plsc.parallel_loop(0, _PPW // 16, unroll=8)
    def _gather_step(i):
        r = i // (W // 16)
        c16 = (i % (W // 16)) * 16
        idx = idx_v[pl.ds(i * 16, 16)]
        for t in range(T):
            vals = plsc.load_gather(bank_v, [idx + (t * E)])
            wm_v[t, r, pl.ds(c16, 16)] = vals

    row0 = (wid % _WPB) * _RPW
    for t in range(T):
        pltpu.sync_copy(wm_v.at[t], wm_hbm.at[b, t, pl.ds(row0, _RPW)])


@functools.cache
def _sc_wm():
    mesh = plsc.VectorSubcoreMesh(core_axis_name="c", subcore_axis_name="s",
                                  num_cores=_NC)
    return pl.kernel(
        _sc_wm_body,
        mesh=mesh,
        out_type=jax.ShapeDtypeStruct((B, T, H, W), jnp.float32),
        scratch_types=[
            pltpu.VMEM((T * E,), jnp.float32),
            pltpu.VMEM((_PPW,), jnp.int32),
            pltpu.VMEM((T, _RPW, W), jnp.float32),
        ],
        compiler_params=pltpu.CompilerParams(needs_layout_passes=False),
    )


_CB = 12
_NCB = C // _CB
_RS = 96
_NS_TC = H // _RS


def _row_strip(x, r0, i):
    g0 = r0 + i - 1
    zrow = jnp.zeros((_CB, 1, W), jnp.float32)
    if g0 < 0:
        return jnp.concatenate([zrow, x[:, 0:_RS - 1]], axis=1)
    if g0 + _RS > H:
        return jnp.concatenate([x[:, g0:H], zrow], axis=1)
    return x[:, g0:g0 + _RS]


def _conv_body(wm_ref, x_ref, o_ref, s0a_ref, s2a_ref, s0b_ref, s2b_ref):
    x = x_ref[0]
    zc1 = jnp.zeros((_RS, 1), jnp.float32)
    zcol = jnp.zeros((_CB, _RS, 1), jnp.float32)
    bufs = ((s0a_ref, s2a_ref), (s0b_ref, s2b_ref))
    bufs[0][0][...] = _row_strip(x, 0, 0)
    bufs[0][1][...] = _row_strip(x, 0, 2)
    for si in range(_NS_TC):
        r0 = si * _RS
        cur, nxt = bufs[si % 2], bufs[(si + 1) % 2]
        if si + 1 < _NS_TC:
            nxt[0][...] = _row_strip(x, r0 + _RS, 0)
            nxt[1][...] = _row_strip(x, r0 + _RS, 2)
        xs = (cur[0][...], x[:, r0:r0 + _RS], cur[1][...])
        acc = None
        for j in range(K):
            cj = None
            for i in range(K):
                wmv = wm_ref[0, i * K + j, r0:r0 + _RS, :]
                if j == 0:
                    wmv = jnp.concatenate([wmv[:, 1:], zc1], axis=1)
                elif j == 2:
                    wmv = jnp.concatenate([zc1, wmv[:, :W - 1]], axis=1)
                term = xs[i] * wmv[None]
                cj = term if cj is None else cj + term
            if j == 0:
                cj = jnp.concatenate([zcol, cj[:, :, :W - 1]], axis=2)
            elif j == 2:
                cj = jnp.concatenate([cj[:, :, 1:], zcol], axis=2)
            acc = cj if acc is None else acc + cj
        o_ref[0, :, r0:r0 + _RS] = acc


def _conv(wm, x):
    return pl.pallas_call(
        _conv_body,
        grid=(B, _NCB),
        in_specs=[
            pl.BlockSpec((1, T, H, W), lambda b, c: (b, 0, 0, 0)),
            pl.BlockSpec((1, _CB, H, W), lambda b, c: (b, c, 0, 0)),
        ],
        out_specs=pl.BlockSpec((1, _CB, H, W), lambda b, c: (b, c, 0, 0)),
        out_shape=jax.ShapeDtypeStruct((B, C, H, W), jnp.float32),
        scratch_shapes=[pltpu.VMEM((_CB, _RS, W), jnp.float32)] * 4,
    )(wm, x)


def kernel(input, kernel_bank, buckets):
    bank_t = jnp.transpose(kernel_bank.reshape(B, E, T), (0, 2, 1)).reshape(B * T * E)
    wm = _sc_wm()(bank_t, buckets.reshape(B * N))
    return _conv(wm, input)
